# baseline (speedup 1.0000x reference)
import jax
import jax.numpy as jnp
from jax.experimental import pallas as pl
from jax.experimental.pallas import tpu as pltpu


def _round_up(x, m):
    return (x + m - 1) // m * m


def _rnn_core(x_ref, wih_ref, whh_ref, brnn_ref, wfc_ref, bfc_ref, out_ref, p_ref):
    s, bc, e = x_ref.shape
    hp = whh_ref.shape[0]

    x2d = x_ref[...].reshape(s * bc, e)
    p_ref[...] = (
        jnp.dot(x2d, wih_ref[...], preferred_element_type=jnp.float32)
        + brnn_ref[...]
    )

    w_hh = whh_ref[...]

    def step(t, h):
        row = pl.multiple_of(t * bc, bc)
        acc = (
            jnp.dot(h, w_hh, preferred_element_type=jnp.float32)
            + p_ref[pl.ds(row, bc), :]
        )
        return jnp.tanh(acc).astype(jnp.bfloat16)

    h_last = jax.lax.fori_loop(
        0, s, step, jnp.zeros((bc, hp), jnp.bfloat16), unroll=True
    )

    out_ref[...] = (
        jnp.dot(h_last, wfc_ref[...], preferred_element_type=jnp.float32)
        + bfc_ref[...]
    )


def kernel(token_ids, emb_table, w_ih, w_hh, b_rnn, w_fc, b_fc):
    B, S = token_ids.shape
    E = emb_table.shape[1]
    H = w_ih.shape[1]
    V = w_fc.shape[1]

    Bp = _round_up(max(B, 16), 16)
    Hp = _round_up(H, 128)
    Vp = _round_up(V, 128)
    Bc = Bp // 2

    x_seq = jnp.take(emb_table, token_ids.T, axis=0).astype(jnp.bfloat16)
    x_seq = jnp.pad(x_seq, ((0, 0), (0, Bp - B), (0, 0)))

    w_ih_p = jnp.pad(w_ih.astype(jnp.bfloat16), ((0, 0), (0, Hp - H)))
    w_hh_p = jnp.pad(w_hh.astype(jnp.bfloat16), ((0, Hp - H), (0, Hp - H)))
    b_rnn_p = jnp.pad(b_rnn.astype(jnp.float32), ((0, 0), (0, Hp - H)))
    w_fc_p = jnp.pad(w_fc.astype(jnp.bfloat16), ((0, Hp - H), (0, Vp - V)))
    b_fc_p = jnp.pad(b_fc.astype(jnp.float32), ((0, 0), (0, Vp - V)))

    grid_spec = pltpu.PrefetchScalarGridSpec(
        num_scalar_prefetch=0,
        grid=(2,),
        in_specs=[
            pl.BlockSpec((S, Bc, E), lambda i: (0, i, 0)),
            pl.BlockSpec((E, Hp), lambda i: (0, 0)),
            pl.BlockSpec((Hp, Hp), lambda i: (0, 0)),
            pl.BlockSpec((1, Hp), lambda i: (0, 0)),
            pl.BlockSpec((Hp, Vp), lambda i: (0, 0)),
            pl.BlockSpec((1, Vp), lambda i: (0, 0)),
        ],
        out_specs=pl.BlockSpec((Bc, Vp), lambda i: (i, 0)),
        scratch_shapes=[pltpu.VMEM((S * Bc, Hp), jnp.float32)],
    )

    out_padded = pl.pallas_call(
        _rnn_core,
        out_shape=jax.ShapeDtypeStruct((Bp, Vp), jnp.float32),
        grid_spec=grid_spec,
        compiler_params=pltpu.CompilerParams(
            dimension_semantics=("parallel",),
        ),
    )(x_seq, w_ih_p, w_hh_p, b_rnn_p, w_fc_p, b_fc_p)

    return out_padded[:B, :V]


# speedup vs baseline: 1.2579x; 1.2579x over previous
import jax
import jax.numpy as jnp
from jax.experimental import pallas as pl
from jax.experimental.pallas import tpu as pltpu


def _round_up(x, m):
    return (x + m - 1) // m * m


def _rnn_core(x_ref, wih_ref, whh_ref, brnn_ref, wfc_ref, bfc_ref, out_ref, p_ref):
    sb, _ = x_ref.shape
    bp = out_ref.shape[0]
    hp = whh_ref.shape[0]
    seq_len = sb // bp

    p_ref[...] = (
        jnp.dot(x_ref[...], wih_ref[...], preferred_element_type=jnp.float32)
        + brnn_ref[...]
    )

    w_hh = whh_ref[...]

    def step(t, h):
        row = pl.multiple_of(t * bp, bp)
        acc = (
            jnp.dot(h, w_hh, preferred_element_type=jnp.float32)
            + p_ref[pl.ds(row, bp), :]
        )
        return jnp.tanh(acc).astype(jnp.bfloat16)

    h_last = jax.lax.fori_loop(
        0, seq_len, step, jnp.zeros((bp, hp), jnp.bfloat16), unroll=True
    )

    out_ref[...] = (
        jnp.dot(h_last, wfc_ref[...], preferred_element_type=jnp.float32)
        + bfc_ref[...]
    )


def kernel(token_ids, emb_table, w_ih, w_hh, b_rnn, w_fc, b_fc):
    B, S = token_ids.shape
    E = emb_table.shape[1]
    H = w_ih.shape[1]
    V = w_fc.shape[1]

    Bp = _round_up(max(B, 8), 8)
    Hp = _round_up(H, 128)
    Vp = _round_up(V, 128)

    x_seq = jnp.take(emb_table, token_ids.T, axis=0).astype(jnp.bfloat16)
    x_seq = jnp.pad(x_seq, ((0, 0), (0, Bp - B), (0, 0)))
    x_flat = x_seq.reshape(S * Bp, E)

    w_ih_p = jnp.pad(w_ih.astype(jnp.bfloat16), ((0, 0), (0, Hp - H)))
    w_hh_p = jnp.pad(w_hh.astype(jnp.bfloat16), ((0, Hp - H), (0, Hp - H)))
    b_rnn_p = jnp.pad(b_rnn.astype(jnp.float32), ((0, 0), (0, Hp - H)))
    w_fc_p = jnp.pad(w_fc.astype(jnp.bfloat16), ((0, Hp - H), (0, Vp - V)))
    b_fc_p = jnp.pad(b_fc.astype(jnp.float32), ((0, 0), (0, Vp - V)))

    grid_spec = pltpu.PrefetchScalarGridSpec(
        num_scalar_prefetch=0,
        grid=(1,),
        in_specs=[
            pl.BlockSpec((S * Bp, E), lambda i: (0, 0)),
            pl.BlockSpec((E, Hp), lambda i: (0, 0)),
            pl.BlockSpec((Hp, Hp), lambda i: (0, 0)),
            pl.BlockSpec((1, Hp), lambda i: (0, 0)),
            pl.BlockSpec((Hp, Vp), lambda i: (0, 0)),
            pl.BlockSpec((1, Vp), lambda i: (0, 0)),
        ],
        out_specs=pl.BlockSpec((Bp, Vp), lambda i: (0, 0)),
        scratch_shapes=[pltpu.VMEM((S * Bp, Hp), jnp.float32)],
    )

    out_padded = pl.pallas_call(
        _rnn_core,
        out_shape=jax.ShapeDtypeStruct((Bp, Vp), jnp.float32),
        grid_spec=grid_spec,
        compiler_params=pltpu.CompilerParams(
            dimension_semantics=("arbitrary",),
        ),
    )(x_flat, w_ih_p, w_hh_p, b_rnn_p, w_fc_p, b_fc_p)

    return out_padded[:B, :V]


# speedup vs baseline: 1.5120x; 1.2020x over previous
import jax
import jax.numpy as jnp
from jax.experimental import pallas as pl
from jax.experimental.pallas import tpu as pltpu


def _round_up(x, m):
    return (x + m - 1) // m * m


def _rnn_core(
    x_ref, wih_ref, whh_ref, brnn_ref, wfc_hbm, bfc_ref, out_ref,
    p_ref, wfc_vmem, dma_sem,
):
    sb, _ = x_ref.shape
    bp = out_ref.shape[0]
    hp = whh_ref.shape[0]
    seq_len = sb // bp
    half = bp // 2

    wfc_copy = pltpu.make_async_copy(wfc_hbm, wfc_vmem, dma_sem)
    wfc_copy.start()

    p_ref[...] = (
        jnp.dot(
            x_ref[...],
            wih_ref[...].astype(jnp.bfloat16),
            preferred_element_type=jnp.float32,
        )
        + brnn_ref[...]
    )

    w_hh = whh_ref[...].astype(jnp.bfloat16)

    def step(t, carry):
        h_a, h_b = carry
        base = pl.multiple_of(t * bp, bp)
        acc_a = (
            jnp.dot(h_a, w_hh, preferred_element_type=jnp.float32)
            + p_ref[pl.ds(base, half), :]
        )
        acc_b = (
            jnp.dot(h_b, w_hh, preferred_element_type=jnp.float32)
            + p_ref[pl.ds(base + half, half), :]
        )
        return (
            jnp.tanh(acc_a).astype(jnp.bfloat16),
            jnp.tanh(acc_b).astype(jnp.bfloat16),
        )

    h0 = jnp.zeros((half, hp), jnp.bfloat16)
    h_a, h_b = jax.lax.fori_loop(0, seq_len, step, (h0, h0), unroll=True)
    h_last = jnp.concatenate([h_a, h_b], axis=0)

    wfc_copy.wait()
    out_ref[...] = (
        jnp.dot(
            h_last,
            wfc_vmem[...].astype(jnp.bfloat16),
            preferred_element_type=jnp.float32,
        )
        + bfc_ref[...]
    )


def kernel(token_ids, emb_table, w_ih, w_hh, b_rnn, w_fc, b_fc):
    B, S = token_ids.shape
    E = emb_table.shape[1]
    H = w_ih.shape[1]
    V = w_fc.shape[1]

    Bp = _round_up(max(B, 16), 16)
    Hp = _round_up(H, 128)
    Vp = _round_up(V, 128)

    x_seq = jnp.take(emb_table, token_ids.T, axis=0).astype(jnp.bfloat16)
    x_seq = jnp.pad(x_seq, ((0, 0), (0, Bp - B), (0, 0)))
    x_flat = x_seq.reshape(S * Bp, E)

    w_ih_p = jnp.pad(w_ih, ((0, 0), (0, Hp - H)))
    w_hh_p = jnp.pad(w_hh, ((0, Hp - H), (0, Hp - H)))
    b_rnn_p = jnp.pad(b_rnn, ((0, 0), (0, Hp - H)))
    w_fc_p = jnp.pad(w_fc, ((0, Hp - H), (0, Vp - V)))
    b_fc_p = jnp.pad(b_fc, ((0, 0), (0, Vp - V)))

    grid_spec = pltpu.PrefetchScalarGridSpec(
        num_scalar_prefetch=0,
        grid=(1,),
        in_specs=[
            pl.BlockSpec((S * Bp, E), lambda i: (0, 0)),
            pl.BlockSpec((E, Hp), lambda i: (0, 0)),
            pl.BlockSpec((Hp, Hp), lambda i: (0, 0)),
            pl.BlockSpec((1, Hp), lambda i: (0, 0)),
            pl.BlockSpec(memory_space=pl.ANY),
            pl.BlockSpec((1, Vp), lambda i: (0, 0)),
        ],
        out_specs=pl.BlockSpec((Bp, Vp), lambda i: (0, 0)),
        scratch_shapes=[
            pltpu.VMEM((S * Bp, Hp), jnp.float32),
            pltpu.VMEM((Hp, Vp), jnp.float32),
            pltpu.SemaphoreType.DMA,
        ],
    )

    out_padded = pl.pallas_call(
        _rnn_core,
        out_shape=jax.ShapeDtypeStruct((Bp, Vp), jnp.float32),
        grid_spec=grid_spec,
        compiler_params=pltpu.CompilerParams(
            dimension_semantics=("arbitrary",),
        ),
    )(x_flat, w_ih_p, w_hh_p, b_rnn_p, w_fc_p, b_fc_p)

    return out_padded[:B, :V]


# speedup vs baseline: 1.6825x; 1.1128x over previous
import jax
import jax.numpy as jnp
from jax.experimental import pallas as pl
from jax.experimental.pallas import tpu as pltpu


def _round_up(x, m):
    return (x + m - 1) // m * m


def _rnn_core(
    tok_ref, emb_ref, wih_ref, whh_ref, brnn_ref, wfc_hbm, bfc_ref, out_ref,
    x_ref, p_ref, wfc_vmem, dma_sem,
):
    sb = x_ref.shape[0]
    bp = out_ref.shape[0]
    hp = whh_ref.shape[0]
    seq_len = sb // bp
    half = bp // 2

    wfc_copy = pltpu.make_async_copy(wfc_hbm, wfc_vmem, dma_sem)
    wfc_copy.start()

    def gather_step(t, _):
        base = t * bp
        for b in range(bp):
            idx = tok_ref[base + b]
            x_ref[pl.ds(base + b, 1), :] = emb_ref[pl.ds(idx, 1), :]
        return 0

    jax.lax.fori_loop(0, seq_len, gather_step, 0, unroll=False)

    p_ref[...] = (
        jnp.dot(
            x_ref[...],
            wih_ref[...].astype(jnp.bfloat16),
            preferred_element_type=jnp.float32,
        )
        + brnn_ref[...]
    )

    w_hh = whh_ref[...].astype(jnp.bfloat16)

    def step(t, carry):
        h_a, h_b = carry
        base = pl.multiple_of(t * bp, bp)
        acc_a = (
            jnp.dot(h_a, w_hh, preferred_element_type=jnp.float32)
            + p_ref[pl.ds(base, half), :]
        )
        acc_b = (
            jnp.dot(h_b, w_hh, preferred_element_type=jnp.float32)
            + p_ref[pl.ds(base + half, half), :]
        )
        return (
            jnp.tanh(acc_a).astype(jnp.bfloat16),
            jnp.tanh(acc_b).astype(jnp.bfloat16),
        )

    h0 = jnp.zeros((half, hp), jnp.bfloat16)
    h_a, h_b = jax.lax.fori_loop(0, seq_len, step, (h0, h0), unroll=True)
    h_last = jnp.concatenate([h_a, h_b], axis=0)

    wfc_copy.wait()
    out_ref[...] = (
        jnp.dot(
            h_last.astype(jnp.float32),
            wfc_vmem[...],
            preferred_element_type=jnp.float32,
        )
        + bfc_ref[...]
    )


def kernel(token_ids, emb_table, w_ih, w_hh, b_rnn, w_fc, b_fc):
    B, S = token_ids.shape
    E = emb_table.shape[1]
    H = w_ih.shape[1]
    V = w_fc.shape[1]

    Bp = _round_up(max(B, 16), 16)
    Hp = _round_up(H, 128)
    Vp = _round_up(V, 128)

    tok = jnp.pad(token_ids.T, ((0, 0), (0, Bp - B))).reshape(S * Bp)

    w_ih_p = jnp.pad(w_ih, ((0, 0), (0, Hp - H)))
    w_hh_p = jnp.pad(w_hh, ((0, Hp - H), (0, Hp - H)))
    b_rnn_p = jnp.pad(b_rnn, ((0, 0), (0, Hp - H)))
    w_fc_p = jnp.pad(w_fc, ((0, Hp - H), (0, Vp - V)))
    b_fc_p = jnp.pad(b_fc, ((0, 0), (0, Vp - V)))

    grid_spec = pltpu.PrefetchScalarGridSpec(
        num_scalar_prefetch=1,
        grid=(1,),
        in_specs=[
            pl.BlockSpec(emb_table.shape, lambda i, *_: (0, 0)),
            pl.BlockSpec((E, Hp), lambda i, *_: (0, 0)),
            pl.BlockSpec((Hp, Hp), lambda i, *_: (0, 0)),
            pl.BlockSpec((1, Hp), lambda i, *_: (0, 0)),
            pl.BlockSpec(memory_space=pl.ANY),
            pl.BlockSpec((1, Vp), lambda i, *_: (0, 0)),
        ],
        out_specs=pl.BlockSpec((Bp, Vp), lambda i, *_: (0, 0)),
        scratch_shapes=[
            pltpu.VMEM((S * Bp, E), jnp.float32),
            pltpu.VMEM((S * Bp, Hp), jnp.float32),
            pltpu.VMEM((Hp, Vp), jnp.float32),
            pltpu.SemaphoreType.DMA,
        ],
    )

    out_padded = pl.pallas_call(
        _rnn_core,
        out_shape=jax.ShapeDtypeStruct((Bp, Vp), jnp.float32),
        grid_spec=grid_spec,
        compiler_params=pltpu.CompilerParams(
            dimension_semantics=("arbitrary",),
        ),
    )(tok, emb_table, w_ih_p, w_hh_p, b_rnn_p, w_fc_p, b_fc_p)

    return out_padded[:B, :V]
